# Initial kernel scaffold; baseline (speedup 1.0000x reference)
import jax
import jax.numpy as jnp
from jax.experimental import pallas as pl


def kernel(x, edge_index, W1, b1, W2, b2, eps):
    raise NotImplementedError("write your pallas kernel here")


# speedup vs baseline: 1.7065x; 1.7065x over previous
import functools

import jax
import jax.numpy as jnp
from jax import lax
from jax.experimental import pallas as pl
from jax.experimental.pallas import tpu as pltpu
from jax.experimental.pallas import tpu_sc as plsc

N_NODES = 10000
N_EDGES = 160000
D = 256

NC = 2
NS = 16
L = 16
NW = NC * NS

RPW = 320
TRASH = RPW
ACCR = RPW + 1
BIG = 2048
E_PAD = 163840
NBIG = E_PAD // BIG
GB = 64
PEND = BIG + GB + L


def _sc_body(x_hbm, src_hbm, dst_hbm, out_hbm, acc_v, rows_v, srcb_v, dstb_v,
             psrc_v, pldst_v, sem):
    c = lax.axis_index("c")
    s = lax.axis_index("s")
    w = s * NC + c
    lo = w * RPW

    @pl.loop(0, ACCR)
    def _(r):
        for j in range(D // L):
            acc_v[r, pl.ds(j * L, L)] = jnp.zeros((L,), jnp.float32)

    def process_block(off):
        pltpu.async_copy(x_hbm.at[psrc_v.at[pl.ds(off, GB)]], rows_v, sem).wait()

        @pl.loop(0, GB)
        def _(e):
            lv = pldst_v[pl.ds(off + e, L)]
            ld = lv[0]
            for j in range(D // L):
                plsc.addupdate(acc_v.at[ld, pl.ds(j * L, L)],
                               rows_v[e, pl.ds(j * L, L)])

    @pl.loop(0, NBIG, init_carry=jnp.int32(0))
    def cnt(big, cin):
        pltpu.sync_copy(src_hbm.at[pl.ds(big * BIG, BIG)], srcb_v)
        pltpu.sync_copy(dst_hbm.at[pl.ds(big * BIG, BIG)], dstb_v)

        @pl.loop(0, BIG // L, init_carry=cin)
        def c2(j, cc):
            d = dstb_v[pl.ds(j * L, L)]
            sv = srcb_v[pl.ds(j * L, L)]
            ld = d - lo
            m = (ld >= 0) & (ld < RPW)
            pos = plsc.cumsum(m.astype(jnp.int32)) - 1 + cc
            plsc.store_scatter(psrc_v, [pos], sv, mask=m)
            plsc.store_scatter(pldst_v, [pos], ld, mask=m)
            return cc + plsc.all_reduce_population_count(m)[0]

        nblk = c2 // GB

        @pl.loop(0, nblk)
        def _(b):
            process_block(b * GB)

        for j in range(GB // L):
            v1 = psrc_v[pl.ds(nblk * GB + j * L, L)]
            psrc_v[pl.ds(j * L, L)] = v1
            v2 = pldst_v[pl.ds(nblk * GB + j * L, L)]
            pldst_v[pl.ds(j * L, L)] = v2
        return c2 - nblk * GB

    @pl.when(cnt > 0)
    def _():
        for j in range(GB // L + 1):
            psrc_v[pl.ds(cnt + j * L, L)] = jnp.zeros((L,), jnp.int32)
            pldst_v[pl.ds(cnt + j * L, L)] = jnp.full((L,), TRASH, jnp.int32)
        process_block(0)

    pltpu.sync_copy(acc_v.at[pl.ds(0, RPW)], out_hbm.at[pl.ds(lo, RPW)])


@functools.cache
def _sc_agg_fn():
    return pl.kernel(
        _sc_body,
        out_type=jax.ShapeDtypeStruct((NW * RPW, D), jnp.float32),
        mesh=plsc.VectorSubcoreMesh(core_axis_name="c", subcore_axis_name="s",
                                    num_cores=NC, num_subcores=NS),
        compiler_params=pltpu.CompilerParams(needs_layout_passes=False),
        scratch_types=[
            pltpu.VMEM((ACCR, D), jnp.float32),
            pltpu.VMEM((GB, D), jnp.float32),
            pltpu.VMEM((BIG,), jnp.int32),
            pltpu.VMEM((BIG,), jnp.int32),
            pltpu.VMEM((PEND,), jnp.int32),
            pltpu.VMEM((PEND,), jnp.int32),
            pltpu.SemaphoreType.DMA,
        ],
    )


ROWS_BLK = 400


def _mlp_body(x_ref, agg_ref, w1_ref, b1_ref, w2_ref, b2_ref, eps_ref, o_ref):
    h = x_ref[...] * eps_ref[0, 0] + agg_ref[...]
    h = jnp.dot(h, w1_ref[...], preferred_element_type=jnp.float32) + b1_ref[...]
    h = jnp.maximum(h, 0.0)
    o_ref[...] = jnp.dot(h, w2_ref[...], preferred_element_type=jnp.float32) + b2_ref[...]


def _mlp(x, agg_pad, W1, b1, W2, b2, scale):
    grid = (N_NODES // ROWS_BLK,)
    return pl.pallas_call(
        _mlp_body,
        grid=grid,
        in_specs=[
            pl.BlockSpec((ROWS_BLK, D), lambda i: (i, 0)),
            pl.BlockSpec((ROWS_BLK, D), lambda i: (i, 0)),
            pl.BlockSpec((D, D), lambda i: (0, 0)),
            pl.BlockSpec((1, D), lambda i: (0, 0)),
            pl.BlockSpec((D, D), lambda i: (0, 0)),
            pl.BlockSpec((1, D), lambda i: (0, 0)),
            pl.BlockSpec((1, 1), lambda i: (0, 0), memory_space=pltpu.SMEM),
        ],
        out_specs=pl.BlockSpec((ROWS_BLK, D), lambda i: (i, 0)),
        out_shape=jax.ShapeDtypeStruct((N_NODES, D), jnp.float32),
    )(x, agg_pad, W1, b1, W2, b2, scale)


def kernel(x, edge_index, W1, b1, W2, b2, eps):
    ei = edge_index.astype(jnp.int32)
    src = jnp.concatenate([ei[0], jnp.zeros((E_PAD - N_EDGES,), jnp.int32)])
    dst = jnp.concatenate([ei[1], jnp.full((E_PAD - N_EDGES,), NW * RPW, jnp.int32)])
    agg_pad = _sc_agg_fn()(x, src, dst)
    scale = jnp.reshape(1.0 + eps, (1, 1)).astype(jnp.float32)
    return _mlp(x, agg_pad, W1, b1.reshape(1, D), W2, b2.reshape(1, D), scale)


# speedup vs baseline: 2.7415x; 1.6065x over previous
import functools

import jax
import jax.numpy as jnp
from jax import lax
from jax.experimental import pallas as pl
from jax.experimental.pallas import tpu as pltpu
from jax.experimental.pallas import tpu_sc as plsc

N_NODES = 10000
N_EDGES = 160000
D = 256

NC = 2
NS = 16
L = 16
NW = NC * NS

RPW = 320
TRASH = RPW
ACCR = RPW + 1
BIG = 2048
E_PAD = 163840
NBIG = E_PAD // BIG
GB = 64
PEND = BIG + GB + L


def _sc_body(x_hbm, src_hbm, dst_hbm, out_hbm, acc_v, rows_v, srcb_v, dstb_v,
             psrc_v, pldst_v, sem):
    c = lax.axis_index("c")
    s = lax.axis_index("s")
    w = s * NC + c
    lo = w * RPW

    @pl.loop(0, ACCR)
    def _(r):
        for j in range(D // L):
            acc_v[r, pl.ds(j * L, L)] = jnp.zeros((L,), jnp.float32)

    def process_block(off):
        pltpu.async_copy(x_hbm.at[psrc_v.at[pl.ds(off, GB)]], rows_v, sem).wait()

        @pl.loop(0, GB, step=2)
        def _(e):
            lv = pldst_v[pl.ds(off + e, L)]
            ld0 = lv[0]
            ld1 = lv[1]
            vals0 = [rows_v[e, pl.ds(j * L, L)] for j in range(D // L)]
            vals1 = [rows_v[e + 1, pl.ds(j * L, L)] for j in range(D // L)]
            for j in range(D // L):
                plsc.addupdate(acc_v.at[ld0, pl.ds(j * L, L)], vals0[j])
            for j in range(D // L):
                plsc.addupdate(acc_v.at[ld1, pl.ds(j * L, L)], vals1[j])

    @pl.loop(0, NBIG, init_carry=jnp.int32(0))
    def cnt(big, cin):
        a1 = pltpu.async_copy(src_hbm.at[pl.ds(big * BIG, BIG)], srcb_v, sem)
        a2 = pltpu.async_copy(dst_hbm.at[pl.ds(big * BIG, BIG)], dstb_v, sem)
        a1.wait()
        a2.wait()

        @pl.loop(0, BIG // L, init_carry=cin)
        def c2(j, cc):
            d = dstb_v[pl.ds(j * L, L)]
            sv = srcb_v[pl.ds(j * L, L)]
            ld = d - lo
            m = (ld >= 0) & (ld < RPW)
            pos = plsc.cumsum(m.astype(jnp.int32)) + (cc - 1)
            plsc.store_scatter(psrc_v, [pos], sv, mask=m)
            plsc.store_scatter(pldst_v, [pos], ld, mask=m)
            return pos[L - 1] + 1

        nblk = c2 // GB

        @pl.loop(0, nblk)
        def _(b):
            process_block(b * GB)

        for j in range(GB // L):
            v1 = psrc_v[pl.ds(nblk * GB + j * L, L)]
            psrc_v[pl.ds(j * L, L)] = v1
            v2 = pldst_v[pl.ds(nblk * GB + j * L, L)]
            pldst_v[pl.ds(j * L, L)] = v2
        return c2 - nblk * GB

    @pl.when(cnt > 0)
    def _():
        for j in range(GB // L + 1):
            psrc_v[pl.ds(cnt + j * L, L)] = jnp.zeros((L,), jnp.int32)
            pldst_v[pl.ds(cnt + j * L, L)] = jnp.full((L,), TRASH, jnp.int32)
        process_block(0)

    pltpu.sync_copy(acc_v.at[pl.ds(0, RPW)], out_hbm.at[pl.ds(lo, RPW)])


@functools.cache
def _sc_agg_fn():
    return pl.kernel(
        _sc_body,
        out_type=jax.ShapeDtypeStruct((NW * RPW, D), jnp.float32),
        mesh=plsc.VectorSubcoreMesh(core_axis_name="c", subcore_axis_name="s",
                                    num_cores=NC, num_subcores=NS),
        compiler_params=pltpu.CompilerParams(needs_layout_passes=False),
        scratch_types=[
            pltpu.VMEM((ACCR, D), jnp.float32),
            pltpu.VMEM((GB, D), jnp.float32),
            pltpu.VMEM((BIG,), jnp.int32),
            pltpu.VMEM((BIG,), jnp.int32),
            pltpu.VMEM((PEND,), jnp.int32),
            pltpu.VMEM((PEND,), jnp.int32),
            pltpu.SemaphoreType.DMA,
        ],
    )


ROWS_BLK = 400


def _mlp_body(x_ref, agg_ref, w1_ref, b1_ref, w2_ref, b2_ref, eps_ref, o_ref):
    h = x_ref[...] * eps_ref[0, 0] + agg_ref[...]
    h = jnp.dot(h, w1_ref[...], preferred_element_type=jnp.float32) + b1_ref[...]
    h = jnp.maximum(h, 0.0)
    o_ref[...] = jnp.dot(h, w2_ref[...], preferred_element_type=jnp.float32) + b2_ref[...]


def _mlp(x, agg_pad, W1, b1, W2, b2, scale):
    grid = (N_NODES // ROWS_BLK,)
    return pl.pallas_call(
        _mlp_body,
        grid=grid,
        in_specs=[
            pl.BlockSpec((ROWS_BLK, D), lambda i: (i, 0)),
            pl.BlockSpec((ROWS_BLK, D), lambda i: (i, 0)),
            pl.BlockSpec((D, D), lambda i: (0, 0)),
            pl.BlockSpec((1, D), lambda i: (0, 0)),
            pl.BlockSpec((D, D), lambda i: (0, 0)),
            pl.BlockSpec((1, D), lambda i: (0, 0)),
            pl.BlockSpec((1, 1), lambda i: (0, 0), memory_space=pltpu.SMEM),
        ],
        out_specs=pl.BlockSpec((ROWS_BLK, D), lambda i: (i, 0)),
        out_shape=jax.ShapeDtypeStruct((N_NODES, D), jnp.float32),
    )(x, agg_pad, W1, b1, W2, b2, scale)


def kernel(x, edge_index, W1, b1, W2, b2, eps):
    ei = edge_index.astype(jnp.int32)
    src = jnp.concatenate([ei[0], jnp.zeros((E_PAD - N_EDGES,), jnp.int32)])
    dst = jnp.concatenate([ei[1], jnp.full((E_PAD - N_EDGES,), NW * RPW, jnp.int32)])
    agg_pad = _sc_agg_fn()(x, src, dst)
    scale = jnp.reshape(1.0 + eps, (1, 1)).astype(jnp.float32)
    return _mlp(x, agg_pad, W1, b1.reshape(1, D), W2, b2.reshape(1, D), scale)


# speedup vs baseline: 3.1464x; 1.1477x over previous
import functools

import jax
import jax.numpy as jnp
from jax import lax
from jax.experimental import pallas as pl
from jax.experimental.pallas import tpu as pltpu
from jax.experimental.pallas import tpu_sc as plsc

N_NODES = 10000
N_EDGES = 160000
D = 256

NC = 2
NS = 16
L = 16
NW = NC * NS

RPW = 320
TRASH = RPW
ACCR = RPW + 1
BIG = 2048
E_PAD = 163840
NBIG = E_PAD // BIG
GB = 64
PEND = BIG + GB + L


def _sc_body(x_hbm, src_hbm, dst_hbm, out_hbm, acc_v, rows_v, srcb_v, dstb_v,
             psrc_v, pldst_v, sem, gsem):
    c = lax.axis_index("c")
    s = lax.axis_index("s")
    w = s * NC + c
    lo = w * RPW

    @pl.loop(0, ACCR)
    def _(r):
        for j in range(D // L):
            acc_v[r, pl.ds(j * L, L)] = jnp.zeros((L,), jnp.float32)

    def issue_gather(off, bm):
        pltpu.async_copy(x_hbm.at[psrc_v.at[pl.ds(off, GB)]],
                         rows_v.at[bm], gsem)

    def wait_gather(bm):
        pltpu.make_async_copy(x_hbm.at[pl.ds(0, GB)], rows_v.at[bm],
                              gsem).wait()

    def accum_block(off, bm):
        @pl.loop(0, GB, step=2)
        def _(e):
            lv = pldst_v[pl.ds(off + e, L)]
            ld0 = lv[0]
            ld1 = lv[1]
            vals0 = [rows_v[bm, e, pl.ds(j * L, L)] for j in range(D // L)]
            vals1 = [rows_v[bm, e + 1, pl.ds(j * L, L)] for j in range(D // L)]
            for j in range(D // L):
                plsc.addupdate(acc_v.at[ld0, pl.ds(j * L, L)], vals0[j])
            for j in range(D // L):
                plsc.addupdate(acc_v.at[ld1, pl.ds(j * L, L)], vals1[j])

    def issue_idx(big):
        bb = big % 2
        pltpu.async_copy(src_hbm.at[pl.ds(big * BIG, BIG)], srcb_v.at[bb], sem)
        pltpu.async_copy(dst_hbm.at[pl.ds(big * BIG, BIG)], dstb_v.at[bb], sem)

    issue_idx(0)

    @pl.loop(0, NBIG, init_carry=jnp.int32(0))
    def cnt(big, cin):
        bb = big % 2
        pltpu.make_async_copy(src_hbm.at[pl.ds(0, BIG)], srcb_v.at[bb], sem).wait()
        pltpu.make_async_copy(dst_hbm.at[pl.ds(0, BIG)], dstb_v.at[bb], sem).wait()

        @pl.when(big + 1 < NBIG)
        def _():
            issue_idx(big + 1)

        @pl.loop(0, BIG // L, init_carry=cin)
        def c2(j, cc):
            d = dstb_v[bb, pl.ds(j * L, L)]
            sv = srcb_v[bb, pl.ds(j * L, L)]
            ld = d - lo
            m = (ld >= 0) & (ld < RPW)
            pos = plsc.cumsum(m.astype(jnp.int32)) + (cc - 1)
            plsc.store_scatter(psrc_v, [pos], sv, mask=m)
            plsc.store_scatter(pldst_v, [pos], ld, mask=m)
            return pos[L - 1] + 1

        nblk = c2 // GB

        @pl.when(nblk > 0)
        def _():
            issue_gather(0, 0)

        @pl.loop(0, nblk)
        def _(b):
            bm = b % 2

            @pl.when(b + 1 < nblk)
            def _():
                issue_gather((b + 1) * GB, (b + 1) % 2)

            wait_gather(bm)
            accum_block(b * GB, bm)

        for j in range(GB // L):
            v1 = psrc_v[pl.ds(nblk * GB + j * L, L)]
            psrc_v[pl.ds(j * L, L)] = v1
            v2 = pldst_v[pl.ds(nblk * GB + j * L, L)]
            pldst_v[pl.ds(j * L, L)] = v2
        return c2 - nblk * GB

    @pl.when(cnt > 0)
    def _():
        for j in range(GB // L + 1):
            psrc_v[pl.ds(cnt + j * L, L)] = jnp.zeros((L,), jnp.int32)
            pldst_v[pl.ds(cnt + j * L, L)] = jnp.full((L,), TRASH, jnp.int32)
        issue_gather(0, 0)
        wait_gather(0)
        accum_block(0, 0)

    pltpu.sync_copy(acc_v.at[pl.ds(0, RPW)], out_hbm.at[pl.ds(lo, RPW)])


@functools.cache
def _sc_agg_fn():
    return pl.kernel(
        _sc_body,
        out_type=jax.ShapeDtypeStruct((NW * RPW, D), jnp.float32),
        mesh=plsc.VectorSubcoreMesh(core_axis_name="c", subcore_axis_name="s",
                                    num_cores=NC, num_subcores=NS),
        compiler_params=pltpu.CompilerParams(needs_layout_passes=False),
        scratch_types=[
            pltpu.VMEM((ACCR, D), jnp.float32),
            pltpu.VMEM((2, GB, D), jnp.float32),
            pltpu.VMEM((2, BIG), jnp.int32),
            pltpu.VMEM((2, BIG), jnp.int32),
            pltpu.VMEM((PEND,), jnp.int32),
            pltpu.VMEM((PEND,), jnp.int32),
            pltpu.SemaphoreType.DMA,
            pltpu.SemaphoreType.DMA,
        ],
    )


ROWS_BLK = 400


def _mlp_body(x_ref, agg_ref, w1_ref, b1_ref, w2_ref, b2_ref, eps_ref, o_ref):
    h = x_ref[...] * eps_ref[0, 0] + agg_ref[...]
    h = jnp.dot(h, w1_ref[...], preferred_element_type=jnp.float32) + b1_ref[...]
    h = jnp.maximum(h, 0.0)
    o_ref[...] = jnp.dot(h, w2_ref[...], preferred_element_type=jnp.float32) + b2_ref[...]


def _mlp(x, agg_pad, W1, b1, W2, b2, scale):
    grid = (N_NODES // ROWS_BLK,)
    return pl.pallas_call(
        _mlp_body,
        grid=grid,
        in_specs=[
            pl.BlockSpec((ROWS_BLK, D), lambda i: (i, 0)),
            pl.BlockSpec((ROWS_BLK, D), lambda i: (i, 0)),
            pl.BlockSpec((D, D), lambda i: (0, 0)),
            pl.BlockSpec((1, D), lambda i: (0, 0)),
            pl.BlockSpec((D, D), lambda i: (0, 0)),
            pl.BlockSpec((1, D), lambda i: (0, 0)),
            pl.BlockSpec((1, 1), lambda i: (0, 0), memory_space=pltpu.SMEM),
        ],
        out_specs=pl.BlockSpec((ROWS_BLK, D), lambda i: (i, 0)),
        out_shape=jax.ShapeDtypeStruct((N_NODES, D), jnp.float32),
    )(x, agg_pad, W1, b1, W2, b2, scale)


def kernel(x, edge_index, W1, b1, W2, b2, eps):
    ei = edge_index.astype(jnp.int32)
    src = jnp.concatenate([ei[0], jnp.zeros((E_PAD - N_EDGES,), jnp.int32)])
    dst = jnp.concatenate([ei[1], jnp.full((E_PAD - N_EDGES,), NW * RPW, jnp.int32)])
    agg_pad = _sc_agg_fn()(x, src, dst)
    scale = jnp.reshape(1.0 + eps, (1, 1)).astype(jnp.float32)
    return _mlp(x, agg_pad, W1, b1.reshape(1, D), W2, b2.reshape(1, D), scale)


# speedup vs baseline: 3.6828x; 1.1705x over previous
import functools

import jax
import jax.numpy as jnp
from jax import lax
from jax.experimental import pallas as pl
from jax.experimental.pallas import tpu as pltpu
from jax.experimental.pallas import tpu_sc as plsc

N_NODES = 10000
N_EDGES = 160000
D = 256

NC = 2
NS = 16
L = 16
NW = NC * NS

RPW = 320
TRASH = RPW
ACCR = RPW + 1
BIG = 2048
E_PAD = 163840
NBIG = E_PAD // BIG
GB = 64
PEND = BIG + GB + L


def _sc_body(x_hbm, src_hbm, dst_hbm, out_hbm, acc_v, rows_v, srcb_v, dstb_v,
             psrc_v, pldst_v, sem, gsem):
    c = lax.axis_index("c")
    s = lax.axis_index("s")
    w = s * NC + c
    lo = w * RPW

    @pl.loop(0, ACCR)
    def _(r):
        for j in range(D // L):
            acc_v[r, pl.ds(j * L, L)] = jnp.zeros((L,), jnp.float32)

    def issue_gather(off, bm):
        pltpu.async_copy(x_hbm.at[psrc_v.at[pl.ds(off, GB)]],
                         rows_v.at[bm], gsem)

    def wait_gather(bm):
        pltpu.make_async_copy(x_hbm.at[pl.ds(0, GB)], rows_v.at[bm],
                              gsem).wait()

    def accum_block(off, bm):
        @pl.loop(0, GB, step=2)
        def _(e):
            lv = pldst_v[pl.ds(off + e, L)]
            ld0 = lv[0]
            ld1 = lv[1]
            vals0 = [rows_v[bm, e, pl.ds(j * L, L)] for j in range(D // L)]
            vals1 = [rows_v[bm, e + 1, pl.ds(j * L, L)] for j in range(D // L)]
            for j in range(D // L):
                plsc.addupdate(acc_v.at[ld0, pl.ds(j * L, L)], vals0[j])
            for j in range(D // L):
                plsc.addupdate(acc_v.at[ld1, pl.ds(j * L, L)], vals1[j])

    def issue_idx(big):
        bb = big % 2
        pltpu.async_copy(src_hbm.at[pl.ds(big * BIG, BIG)], srcb_v.at[bb], sem)
        pltpu.async_copy(dst_hbm.at[pl.ds(big * BIG, BIG)], dstb_v.at[bb], sem)

    issue_idx(0)

    @pl.loop(0, NBIG, init_carry=jnp.int32(0))
    def cnt(big, cin):
        bb = big % 2
        pltpu.make_async_copy(src_hbm.at[pl.ds(0, BIG)], srcb_v.at[bb], sem).wait()
        pltpu.make_async_copy(dst_hbm.at[pl.ds(0, BIG)], dstb_v.at[bb], sem).wait()

        @pl.when(big + 1 < NBIG)
        def _():
            issue_idx(big + 1)

        @pl.loop(0, BIG // (L * 4), init_carry=cin)
        def c2(j4, cc):
            data = []
            for u in range(4):
                d = dstb_v[bb, pl.ds((j4 * 4 + u) * L, L)]
                sv = srcb_v[bb, pl.ds((j4 * 4 + u) * L, L)]
                ld = d - lo
                m = (ld >= 0) & (ld < RPW)
                sc = plsc.cumsum(m.astype(jnp.int32))
                data.append((sv, ld, m, sc))
            tot = cc
            for sv, ld, m, sc in data:
                pos = sc + (tot - 1)
                plsc.store_scatter(psrc_v, [pos], sv, mask=m)
                plsc.store_scatter(pldst_v, [pos], ld, mask=m)
                tot = tot + sc[L - 1]
            return tot

        nblk = c2 // GB

        @pl.when(nblk > 0)
        def _():
            issue_gather(0, 0)

        @pl.loop(0, nblk)
        def _(b):
            bm = b % 2

            @pl.when(b + 1 < nblk)
            def _():
                issue_gather((b + 1) * GB, (b + 1) % 2)

            wait_gather(bm)
            accum_block(b * GB, bm)

        for j in range(GB // L):
            v1 = psrc_v[pl.ds(nblk * GB + j * L, L)]
            psrc_v[pl.ds(j * L, L)] = v1
            v2 = pldst_v[pl.ds(nblk * GB + j * L, L)]
            pldst_v[pl.ds(j * L, L)] = v2
        return c2 - nblk * GB

    @pl.when(cnt > 0)
    def _():
        for j in range(GB // L + 1):
            psrc_v[pl.ds(cnt + j * L, L)] = jnp.zeros((L,), jnp.int32)
            pldst_v[pl.ds(cnt + j * L, L)] = jnp.full((L,), TRASH, jnp.int32)
        issue_gather(0, 0)
        wait_gather(0)
        accum_block(0, 0)

    pltpu.sync_copy(acc_v.at[pl.ds(0, RPW)], out_hbm.at[pl.ds(lo, RPW)])


@functools.cache
def _sc_agg_fn():
    return pl.kernel(
        _sc_body,
        out_type=jax.ShapeDtypeStruct((NW * RPW, D), jnp.float32),
        mesh=plsc.VectorSubcoreMesh(core_axis_name="c", subcore_axis_name="s",
                                    num_cores=NC, num_subcores=NS),
        compiler_params=pltpu.CompilerParams(needs_layout_passes=False),
        scratch_types=[
            pltpu.VMEM((ACCR, D), jnp.float32),
            pltpu.VMEM((2, GB, D), jnp.float32),
            pltpu.VMEM((2, BIG), jnp.int32),
            pltpu.VMEM((2, BIG), jnp.int32),
            pltpu.VMEM((PEND,), jnp.int32),
            pltpu.VMEM((PEND,), jnp.int32),
            pltpu.SemaphoreType.DMA,
            pltpu.SemaphoreType.DMA,
        ],
    )


ROWS_BLK = 400


def _mlp_body(x_ref, agg_ref, w1_ref, b1_ref, w2_ref, b2_ref, eps_ref, o_ref):
    h = x_ref[...] * eps_ref[0, 0] + agg_ref[...]
    h = jnp.dot(h, w1_ref[...], preferred_element_type=jnp.float32) + b1_ref[...]
    h = jnp.maximum(h, 0.0)
    o_ref[...] = jnp.dot(h, w2_ref[...], preferred_element_type=jnp.float32) + b2_ref[...]


def _mlp(x, agg_pad, W1, b1, W2, b2, scale):
    grid = (N_NODES // ROWS_BLK,)
    return pl.pallas_call(
        _mlp_body,
        grid=grid,
        in_specs=[
            pl.BlockSpec((ROWS_BLK, D), lambda i: (i, 0)),
            pl.BlockSpec((ROWS_BLK, D), lambda i: (i, 0)),
            pl.BlockSpec((D, D), lambda i: (0, 0)),
            pl.BlockSpec((1, D), lambda i: (0, 0)),
            pl.BlockSpec((D, D), lambda i: (0, 0)),
            pl.BlockSpec((1, D), lambda i: (0, 0)),
            pl.BlockSpec((1, 1), lambda i: (0, 0), memory_space=pltpu.SMEM),
        ],
        out_specs=pl.BlockSpec((ROWS_BLK, D), lambda i: (i, 0)),
        out_shape=jax.ShapeDtypeStruct((N_NODES, D), jnp.float32),
    )(x, agg_pad, W1, b1, W2, b2, scale)


def kernel(x, edge_index, W1, b1, W2, b2, eps):
    ei = edge_index.astype(jnp.int32)
    src = jnp.concatenate([ei[0], jnp.zeros((E_PAD - N_EDGES,), jnp.int32)])
    dst = jnp.concatenate([ei[1], jnp.full((E_PAD - N_EDGES,), NW * RPW, jnp.int32)])
    agg_pad = _sc_agg_fn()(x, src, dst)
    scale = jnp.reshape(1.0 + eps, (1, 1)).astype(jnp.float32)
    return _mlp(x, agg_pad, W1, b1.reshape(1, D), W2, b2.reshape(1, D), scale)


# speedup vs baseline: 3.8818x; 1.0540x over previous
import functools

import jax
import jax.numpy as jnp
from jax import lax
from jax.experimental import pallas as pl
from jax.experimental.pallas import tpu as pltpu
from jax.experimental.pallas import tpu_sc as plsc

N_NODES = 10000
N_EDGES = 160000
D = 256

NC = 2
NS = 16
L = 16
NW = NC * NS

RPW = 320
TRASH = RPW
ACCR = RPW + 1
BIG = 2048
E_PAD = 163840
NBIG = E_PAD // BIG
GB = 64
PEND = BIG + GB + L


def _sc_body(x_hbm, src_hbm, dst_hbm, out_hbm, acc_v, rows_v, srcb_v, dstb_v,
             psrc_v, pldst_v, sem, gsem):
    c = lax.axis_index("c")
    s = lax.axis_index("s")
    w = s * NC + c
    lo = w * RPW

    @pl.loop(0, ACCR)
    def _(r):
        for j in range(D // L):
            acc_v[r, pl.ds(j * L, L)] = jnp.zeros((L,), jnp.float32)

    def issue_gather(off, bm):
        pltpu.async_copy(x_hbm.at[psrc_v.at[pl.ds(off, GB)]],
                         rows_v.at[bm], gsem)

    def wait_gather(bm):
        pltpu.make_async_copy(x_hbm.at[pl.ds(0, GB)], rows_v.at[bm],
                              gsem).wait()

    def accum_block(off, bm):
        @pl.loop(0, GB, step=2)
        def _(e):
            lv = pldst_v[pl.ds(off + e, L)]
            ld0 = lv[0]
            ld1 = lv[1]
            vals0 = [rows_v[bm, e, pl.ds(j * L, L)] for j in range(D // L)]
            vals1 = [rows_v[bm, e + 1, pl.ds(j * L, L)] for j in range(D // L)]
            for j in range(D // L):
                plsc.addupdate(acc_v.at[ld0, pl.ds(j * L, L)], vals0[j])
            for j in range(D // L):
                plsc.addupdate(acc_v.at[ld1, pl.ds(j * L, L)], vals1[j])

    def issue_idx(big):
        bb = big % 2
        pltpu.async_copy(src_hbm.at[pl.ds(big * BIG, BIG)], srcb_v.at[bb], sem)
        pltpu.async_copy(dst_hbm.at[pl.ds(big * BIG, BIG)], dstb_v.at[bb], sem)

    issue_idx(0)

    @pl.loop(0, NBIG, init_carry=jnp.int32(0))
    def cnt(big, cin):
        bb = big % 2
        pltpu.make_async_copy(src_hbm.at[pl.ds(0, BIG)], srcb_v.at[bb], sem).wait()
        pltpu.make_async_copy(dst_hbm.at[pl.ds(0, BIG)], dstb_v.at[bb], sem).wait()

        @pl.when(big + 1 < NBIG)
        def _():
            issue_idx(big + 1)

        def scan_range(j_lo, j_hi, cc0):
            @pl.loop(j_lo, j_hi, init_carry=cc0)
            def cres(j4, cc):
                data = []
                for u in range(4):
                    d = dstb_v[bb, pl.ds((j4 * 4 + u) * L, L)]
                    sv = srcb_v[bb, pl.ds((j4 * 4 + u) * L, L)]
                    ld = d - lo
                    m = (ld >= 0) & (ld < RPW)
                    sc = plsc.cumsum(m.astype(jnp.int32))
                    data.append((sv, ld, m, sc))
                tot = cc
                for sv, ld, m, sc in data:
                    pos = sc + (tot - 1)
                    plsc.store_scatter(psrc_v, [pos], sv, mask=m)
                    plsc.store_scatter(pldst_v, [pos], ld, mask=m)
                    tot = tot + sc[L - 1]
                return tot
            return cres

        NJ4 = BIG // (L * 4)
        c1 = scan_range(0, NJ4 // 2, cin)
        nb1 = jnp.minimum(c1 // GB, 2)

        @pl.when(nb1 >= 1)
        def _():
            issue_gather(0, 0)

        @pl.when(nb1 >= 2)
        def _():
            issue_gather(GB, 1)

        c2 = scan_range(NJ4 // 2, NJ4, c1)
        nblk = c2 // GB

        @pl.loop(0, nblk, init_carry=nb1)
        def _(b, iss):
            tgt = jnp.minimum(b + 2, nblk)

            @pl.loop(iss, tgt)
            def _(k):
                issue_gather(k * GB, k % 2)

            wait_gather(b % 2)
            accum_block(b * GB, b % 2)
            return tgt

        for j in range(GB // L):
            v1 = psrc_v[pl.ds(nblk * GB + j * L, L)]
            psrc_v[pl.ds(j * L, L)] = v1
            v2 = pldst_v[pl.ds(nblk * GB + j * L, L)]
            pldst_v[pl.ds(j * L, L)] = v2
        return c2 - nblk * GB

    @pl.when(cnt > 0)
    def _():
        for j in range(GB // L + 1):
            psrc_v[pl.ds(cnt + j * L, L)] = jnp.zeros((L,), jnp.int32)
            pldst_v[pl.ds(cnt + j * L, L)] = jnp.full((L,), TRASH, jnp.int32)
        issue_gather(0, 0)
        wait_gather(0)
        accum_block(0, 0)

    pltpu.sync_copy(acc_v.at[pl.ds(0, RPW)], out_hbm.at[pl.ds(lo, RPW)])


@functools.cache
def _sc_agg_fn():
    return pl.kernel(
        _sc_body,
        out_type=jax.ShapeDtypeStruct((NW * RPW, D), jnp.float32),
        mesh=plsc.VectorSubcoreMesh(core_axis_name="c", subcore_axis_name="s",
                                    num_cores=NC, num_subcores=NS),
        compiler_params=pltpu.CompilerParams(needs_layout_passes=False),
        scratch_types=[
            pltpu.VMEM((ACCR, D), jnp.float32),
            pltpu.VMEM((2, GB, D), jnp.float32),
            pltpu.VMEM((2, BIG), jnp.int32),
            pltpu.VMEM((2, BIG), jnp.int32),
            pltpu.VMEM((PEND,), jnp.int32),
            pltpu.VMEM((PEND,), jnp.int32),
            pltpu.SemaphoreType.DMA,
            pltpu.SemaphoreType.DMA,
        ],
    )


ROWS_BLK = 400


def _mlp_body(x_ref, agg_ref, w1_ref, b1_ref, w2_ref, b2_ref, eps_ref, o_ref):
    h = x_ref[...] * eps_ref[0, 0] + agg_ref[...]
    h = jnp.dot(h, w1_ref[...], preferred_element_type=jnp.float32) + b1_ref[...]
    h = jnp.maximum(h, 0.0)
    o_ref[...] = jnp.dot(h, w2_ref[...], preferred_element_type=jnp.float32) + b2_ref[...]


def _mlp(x, agg_pad, W1, b1, W2, b2, scale):
    grid = (N_NODES // ROWS_BLK,)
    return pl.pallas_call(
        _mlp_body,
        grid=grid,
        in_specs=[
            pl.BlockSpec((ROWS_BLK, D), lambda i: (i, 0)),
            pl.BlockSpec((ROWS_BLK, D), lambda i: (i, 0)),
            pl.BlockSpec((D, D), lambda i: (0, 0)),
            pl.BlockSpec((1, D), lambda i: (0, 0)),
            pl.BlockSpec((D, D), lambda i: (0, 0)),
            pl.BlockSpec((1, D), lambda i: (0, 0)),
            pl.BlockSpec((1, 1), lambda i: (0, 0), memory_space=pltpu.SMEM),
        ],
        out_specs=pl.BlockSpec((ROWS_BLK, D), lambda i: (i, 0)),
        out_shape=jax.ShapeDtypeStruct((N_NODES, D), jnp.float32),
    )(x, agg_pad, W1, b1, W2, b2, scale)


def kernel(x, edge_index, W1, b1, W2, b2, eps):
    ei = edge_index.astype(jnp.int32)
    src = jnp.concatenate([ei[0], jnp.zeros((E_PAD - N_EDGES,), jnp.int32)])
    dst = jnp.concatenate([ei[1], jnp.full((E_PAD - N_EDGES,), NW * RPW, jnp.int32)])
    agg_pad = _sc_agg_fn()(x, src, dst)
    scale = jnp.reshape(1.0 + eps, (1, 1)).astype(jnp.float32)
    return _mlp(x, agg_pad, W1, b1.reshape(1, D), W2, b2.reshape(1, D), scale)


# speedup vs baseline: 4.0621x; 1.0464x over previous
import functools

import jax
import jax.numpy as jnp
from jax import lax
from jax.experimental import pallas as pl
from jax.experimental.pallas import tpu as pltpu
from jax.experimental.pallas import tpu_sc as plsc

N_NODES = 10000
N_EDGES = 160000
D = 256

NC = 2
NS = 16
L = 16
NW = NC * NS

RPW = 320
TRASH = RPW
ACCR = RPW + 1
BIG = 2048
E_PAD = 163840
NBIG = E_PAD // BIG
GB = 64
PEND = BIG + GB + L


def _sc_body(x_hbm, src_hbm, dst_hbm, out_hbm, acc_v, rows_v, srcb_v, dstb_v,
             psrc_v, pldst_v, sem, gsem):
    c = lax.axis_index("c")
    s = lax.axis_index("s")
    w = s * NC + c
    lo = w * RPW

    @pl.loop(0, ACCR)
    def _(r):
        for j in range(D // L):
            acc_v[r, pl.ds(j * L, L)] = jnp.zeros((L,), jnp.float32)

    def issue_gather(off, bm):
        pltpu.async_copy(x_hbm.at[psrc_v.at[pl.ds(off, GB)]],
                         rows_v.at[bm], gsem)

    def wait_gather(bm):
        pltpu.make_async_copy(x_hbm.at[pl.ds(0, GB)], rows_v.at[bm],
                              gsem).wait()

    def accum_block(off, bm):
        @pl.loop(0, GB, step=2)
        def _(e):
            lv = pldst_v[pl.ds(off + e, L)]
            ld0 = lv[0]
            ld1 = lv[1]
            vals0 = [rows_v[bm, e, pl.ds(j * L, L)] for j in range(D // L)]
            vals1 = [rows_v[bm, e + 1, pl.ds(j * L, L)] for j in range(D // L)]
            for j in range(D // L):
                plsc.addupdate(acc_v.at[ld0, pl.ds(j * L, L)], vals0[j])
            for j in range(D // L):
                plsc.addupdate(acc_v.at[ld1, pl.ds(j * L, L)], vals1[j])

    def issue_idx(big):
        bb = big % 2
        pltpu.async_copy(src_hbm.at[pl.ds(big * BIG, BIG)], srcb_v.at[bb], sem)
        pltpu.async_copy(dst_hbm.at[pl.ds(big * BIG, BIG)], dstb_v.at[bb], sem)

    issue_idx(0)

    @pl.loop(0, NBIG, init_carry=jnp.int32(0))
    def cnt(big, cin):
        bb = big % 2
        pltpu.make_async_copy(src_hbm.at[pl.ds(0, BIG)], srcb_v.at[bb], sem).wait()
        pltpu.make_async_copy(dst_hbm.at[pl.ds(0, BIG)], dstb_v.at[bb], sem).wait()

        @pl.when(big + 1 < NBIG)
        def _():
            issue_idx(big + 1)

        def scan_range(j_lo, j_hi, cc0):
            @pl.loop(j_lo, j_hi, init_carry=cc0)
            def cres(j8, cc):
                data = []
                for u in range(8):
                    d = dstb_v[bb, pl.ds((j8 * 8 + u) * L, L)]
                    sv = srcb_v[bb, pl.ds((j8 * 8 + u) * L, L)]
                    ld = d - lo
                    m = (ld >= 0) & (ld < RPW)
                    sc = plsc.cumsum(m.astype(jnp.int32))
                    data.append((sv, ld, m, sc))
                tot = cc
                for sv, ld, m, sc in data:
                    pos = sc + (tot - 1)
                    plsc.store_scatter(psrc_v, [pos], sv, mask=m)
                    plsc.store_scatter(pldst_v, [pos], ld, mask=m)
                    tot = tot + sc[L - 1]
                return tot
            return cres

        NJ4 = BIG // (L * 8)
        c1 = scan_range(0, NJ4 // 2, cin)
        nb1 = jnp.minimum(c1 // GB, 2)

        @pl.when(nb1 >= 1)
        def _():
            issue_gather(0, 0)

        @pl.when(nb1 >= 2)
        def _():
            issue_gather(GB, 1)

        c2 = scan_range(NJ4 // 2, NJ4, c1)
        nblk = c2 // GB

        @pl.loop(0, nblk, init_carry=nb1)
        def _(b, iss):
            tgt = jnp.minimum(b + 2, nblk)

            @pl.loop(iss, tgt)
            def _(k):
                issue_gather(k * GB, k % 2)

            wait_gather(b % 2)
            accum_block(b * GB, b % 2)
            return tgt

        for j in range(GB // L):
            v1 = psrc_v[pl.ds(nblk * GB + j * L, L)]
            psrc_v[pl.ds(j * L, L)] = v1
            v2 = pldst_v[pl.ds(nblk * GB + j * L, L)]
            pldst_v[pl.ds(j * L, L)] = v2
        return c2 - nblk * GB

    @pl.when(cnt > 0)
    def _():
        for j in range(GB // L + 1):
            psrc_v[pl.ds(cnt + j * L, L)] = jnp.zeros((L,), jnp.int32)
            pldst_v[pl.ds(cnt + j * L, L)] = jnp.full((L,), TRASH, jnp.int32)
        issue_gather(0, 0)
        wait_gather(0)
        accum_block(0, 0)

    pltpu.sync_copy(acc_v.at[pl.ds(0, RPW)], out_hbm.at[pl.ds(lo, RPW)])


@functools.cache
def _sc_agg_fn():
    return pl.kernel(
        _sc_body,
        out_type=jax.ShapeDtypeStruct((NW * RPW, D), jnp.float32),
        mesh=plsc.VectorSubcoreMesh(core_axis_name="c", subcore_axis_name="s",
                                    num_cores=NC, num_subcores=NS),
        compiler_params=pltpu.CompilerParams(needs_layout_passes=False),
        scratch_types=[
            pltpu.VMEM((ACCR, D), jnp.float32),
            pltpu.VMEM((2, GB, D), jnp.float32),
            pltpu.VMEM((2, BIG), jnp.int32),
            pltpu.VMEM((2, BIG), jnp.int32),
            pltpu.VMEM((PEND,), jnp.int32),
            pltpu.VMEM((PEND,), jnp.int32),
            pltpu.SemaphoreType.DMA,
            pltpu.SemaphoreType.DMA,
        ],
    )


ROWS_BLK = 400


def _mlp_body(x_ref, agg_ref, w1_ref, b1_ref, w2_ref, b2_ref, eps_ref, o_ref):
    h = x_ref[...] * eps_ref[0, 0] + agg_ref[...]
    h = jnp.dot(h, w1_ref[...], preferred_element_type=jnp.float32) + b1_ref[...]
    h = jnp.maximum(h, 0.0)
    o_ref[...] = jnp.dot(h, w2_ref[...], preferred_element_type=jnp.float32) + b2_ref[...]


def _mlp(x, agg_pad, W1, b1, W2, b2, scale):
    grid = (N_NODES // ROWS_BLK,)
    return pl.pallas_call(
        _mlp_body,
        grid=grid,
        in_specs=[
            pl.BlockSpec((ROWS_BLK, D), lambda i: (i, 0)),
            pl.BlockSpec((ROWS_BLK, D), lambda i: (i, 0)),
            pl.BlockSpec((D, D), lambda i: (0, 0)),
            pl.BlockSpec((1, D), lambda i: (0, 0)),
            pl.BlockSpec((D, D), lambda i: (0, 0)),
            pl.BlockSpec((1, D), lambda i: (0, 0)),
            pl.BlockSpec((1, 1), lambda i: (0, 0), memory_space=pltpu.SMEM),
        ],
        out_specs=pl.BlockSpec((ROWS_BLK, D), lambda i: (i, 0)),
        out_shape=jax.ShapeDtypeStruct((N_NODES, D), jnp.float32),
    )(x, agg_pad, W1, b1, W2, b2, scale)


def kernel(x, edge_index, W1, b1, W2, b2, eps):
    ei = edge_index.astype(jnp.int32)
    src = jnp.concatenate([ei[0], jnp.zeros((E_PAD - N_EDGES,), jnp.int32)])
    dst = jnp.concatenate([ei[1], jnp.full((E_PAD - N_EDGES,), NW * RPW, jnp.int32)])
    agg_pad = _sc_agg_fn()(x, src, dst)
    scale = jnp.reshape(1.0 + eps, (1, 1)).astype(jnp.float32)
    return _mlp(x, agg_pad, W1, b1.reshape(1, D), W2, b2.reshape(1, D), scale)
